# baseline (speedup 1.0000x reference)
import functools
import jax
import jax.numpy as jnp
from jax import lax
from jax.experimental import pallas as pl
from jax.experimental.pallas import tpu as pltpu
from jax.experimental.pallas import tpu_sc as plsc

_D = 64
_BW = 128
_T = 200


@functools.cache
def _build(B4096, V):
    nw = 32
    info = plsc.get_sparse_core_info()
    nc = info.num_cores
    mesh = plsc.VectorSubcoreMesh(core_axis_name="c", subcore_axis_name="s")

    @functools.partial(
        pl.kernel,
        mesh=mesh,
        out_type=jax.ShapeDtypeStruct((_T, _D, B4096), jnp.float32),
        scratch_types=[
            pltpu.VMEM((_T, _BW), jnp.int32),
            pltpu.VMEM((4, _BW), jnp.int32),
            pltpu.VMEM((4, _BW, _BW), jnp.float32),
            pltpu.VMEM((4, _BW), jnp.float32),
            pltpu.VMEM((2, _D, _BW), jnp.float32),
            pltpu.SemaphoreType.DMA,
            pltpu.SemaphoreType.DMA,
            pltpu.SemaphoreType.DMA,
            pltpu.SemaphoreType.DMA,
            pltpu.SemaphoreType.DMA,
            pltpu.SemaphoreType.DMA,
            pltpu.SemaphoreType.DMA,
            pltpu.SemaphoreType.DMA,
            pltpu.SemaphoreType.DMA,
            pltpu.SemaphoreType.DMA,
        ],
        compiler_params=pltpu.CompilerParams(needs_layout_passes=False),
    )
    def gather_kernel(xt_hbm, w2_hbm, m_hbm, out_hbm, idx_v, pidx_v, gath_v,
                      mv_v, ob_v, sg0, sg1, sg2, sg3, sm0, sm1, sm2, sm3,
                      so0, so1):
        wid = lax.axis_index("s") * nc + lax.axis_index("c")
        b0 = pl.multiple_of(wid * _BW, _BW)
        sem_g = (sg0, sg1, sg2, sg3)
        sem_m = (sm0, sm1, sm2, sm3)
        sem_o = (so0, so1)

        pltpu.sync_copy(xt_hbm.at[:, pl.ds(b0, _BW)], idx_v)

        def prep(t, slot):
            @plsc.parallel_loop(0, _BW // 16)
            def _(j):
                sl = pl.ds(j * 16, 16)
                pidx_v[slot, sl] = lax.shift_right_logical(idx_v[t, sl], 1)

        def fire(t, slot):
            pltpu.async_copy(w2_hbm.at[pidx_v.at[slot]], gath_v.at[slot],
                             sem_g[slot])
            pltpu.async_copy(m_hbm.at[idx_v.at[t]], mv_v.at[slot],
                             sem_m[slot])

        def wait_gather(t, slot):
            pltpu.make_async_copy(w2_hbm.at[pidx_v.at[slot]],
                                  gath_v.at[slot], sem_g[slot]).wait()
            pltpu.make_async_copy(m_hbm.at[idx_v.at[t]], mv_v.at[slot],
                                  sem_m[slot]).wait()

        def drain_out(t, slot):
            pltpu.make_async_copy(ob_v.at[slot],
                                  out_hbm.at[t, :, pl.ds(b0, _BW)],
                                  sem_o[slot]).wait()

        def compute(t, gslot, slot):
            lanes = lax.iota(jnp.int32, 16)

            @plsc.parallel_loop(0, _BW // 32)
            def _(j):
                sla = pl.ds(j * 16, 16)
                slb = pl.ds(j * 16 + 64, 16)
                rva = j * 16 + lanes
                rvb = rva + 64
                cba = (idx_v[t, sla] & 1) * _D
                cbb = (idx_v[t, slb] & 1) * _D
                mva = mv_v[gslot, sla]
                mvb = mv_v[gslot, slb]
                for d in range(_D):
                    va = plsc.load_gather(gath_v.at[gslot], [rva, cba + d])
                    vb = plsc.load_gather(gath_v.at[gslot], [rvb, cbb + d])
                    ob_v[slot, d, sla] = va * mva
                    ob_v[slot, d, slb] = vb * mvb

        for t0 in (0, 1):
            prep(t0, t0)
            fire(t0, t0)

        def quad(i, carry):
            for k in (0, 1, 2, 3):
                t = 4 * i + k
                gs = k
                obs = k % 2
                nxt = (k + 2) % 4
                if k < 2:
                    prep(t + 2, nxt)
                    fire(t + 2, nxt)
                else:
                    @pl.when(t + 2 < _T)
                    def _():
                        prep(t + 2, nxt)
                        fire(t + 2, nxt)
                wait_gather(t, gs)
                if k < 2:
                    @pl.when(i >= 1)
                    def _():
                        drain_out(t - 2, obs)
                else:
                    drain_out(t - 2, obs)
                compute(t, gs, obs)
                pltpu.async_copy(ob_v.at[obs],
                                 out_hbm.at[t, :, pl.ds(b0, _BW)],
                                 sem_o[obs])
            return carry

        lax.fori_loop(0, _T // 4, quad, 0)
        drain_out(_T - 2, 0)
        drain_out(_T - 1, 1)

    return gather_kernel


def kernel(x, W, mask):
    V = W.shape[0]
    xt = x.T
    w2 = W.reshape(V // 2, 2 * _D)
    mf = mask.reshape(V)
    out_t = _build(x.shape[0], V)(xt, w2, mf)
    return out_t.transpose(2, 0, 1)


# speedup vs baseline: 1.2789x; 1.2789x over previous
import functools
import jax
import jax.numpy as jnp
from jax import lax
from jax.experimental import pallas as pl
from jax.experimental.pallas import tpu as pltpu
from jax.experimental.pallas import tpu_sc as plsc

_D = 64
_CH = 128
_T = 200


@functools.cache
def _build(B, V):
    info = plsc.get_sparse_core_info()
    nc = info.num_cores
    mesh = plsc.VectorSubcoreMesh(core_axis_name="c", subcore_axis_name="s")

    @functools.partial(
        pl.kernel,
        mesh=mesh,
        out_type=jax.ShapeDtypeStruct((B, _D), jnp.float32),
        scratch_types=[
            pltpu.VMEM((_T, _CH), jnp.int32),
            pltpu.VMEM((4, _CH, _D), jnp.float32),
            pltpu.VMEM((4, _CH), jnp.float32),
            pltpu.SemaphoreType.DMA,
            pltpu.SemaphoreType.DMA,
            pltpu.SemaphoreType.DMA,
            pltpu.SemaphoreType.DMA,
            pltpu.SemaphoreType.DMA,
            pltpu.SemaphoreType.DMA,
            pltpu.SemaphoreType.DMA,
            pltpu.SemaphoreType.DMA,
            pltpu.SemaphoreType.DMA,
            pltpu.SemaphoreType.DMA,
            pltpu.SemaphoreType.DMA,
            pltpu.SemaphoreType.DMA,
        ],
        compiler_params=pltpu.CompilerParams(use_tc_tiling_on_sc=False,
                                             needs_layout_passes=False),
    )
    def gather_kernel(x_hbm, w_hbm, m_hbm, out_hbm, idx_v, gath_v, mv_v,
                      sg0, sg1, sg2, sg3, sm0, sm1, sm2, sm3,
                      so0, so1, so2, so3):
        wid = lax.axis_index("s") * nc + lax.axis_index("c")
        base = pl.multiple_of(wid * (_T * _CH), _T * _CH)
        xrow0 = pl.multiple_of(wid * _T, 8)
        sem_g = (sg0, sg1, sg2, sg3)
        sem_m = (sm0, sm1, sm2, sm3)
        sem_o = (so0, so1, so2, so3)

        pltpu.sync_copy(x_hbm.at[pl.ds(xrow0, _T)], idx_v)

        def fire(t, slot):
            pltpu.async_copy(w_hbm.at[idx_v.at[t]], gath_v.at[slot],
                             sem_g[slot])
            pltpu.async_copy(m_hbm.at[idx_v.at[t]], mv_v.at[slot],
                             sem_m[slot])

        def wait_gather(t, slot):
            pltpu.make_async_copy(w_hbm.at[idx_v.at[t]], gath_v.at[slot],
                                  sem_g[slot]).wait()
            pltpu.make_async_copy(m_hbm.at[idx_v.at[t]], mv_v.at[slot],
                                  sem_m[slot]).wait()

        def out_ref(t):
            row0 = pl.multiple_of(base + t * _CH, _CH)
            return out_hbm.at[pl.ds(row0, _CH)]

        def drain_out(t, slot):
            pltpu.make_async_copy(gath_v.at[slot], out_ref(t),
                                  sem_o[slot]).wait()

        def compute(t, slot):
            @plsc.parallel_loop(0, _CH // 16)
            def _(j):
                mvs = mv_v[slot, pl.ds(j * 16, 16)]
                for i in range(16):
                    m = mvs[i]
                    r = j * 16 + i
                    for g in range(_D // 16):
                        sl = pl.ds(g * 16, 16)
                        gath_v[slot, r, sl] = gath_v[slot, r, sl] * m

        for t0 in (0, 1):
            fire(t0, t0)

        def quad(i, carry):
            for k in (0, 1, 2, 3):
                t = 4 * i + k
                s = k
                nxt = (k + 2) % 4
                if k < 2:
                    @pl.when(i >= 1)
                    def _():
                        drain_out(t - 2, nxt)
                    fire(t + 2, nxt)
                else:
                    drain_out(t - 2, nxt)

                    @pl.when(t + 2 < _T)
                    def _():
                        fire(t + 2, nxt)
                wait_gather(t, s)
                compute(t, s)
                pltpu.async_copy(gath_v.at[s], out_ref(t), sem_o[s])
            return carry

        lax.fori_loop(0, _T // 4, quad, 0)
        drain_out(_T - 2, 2)
        drain_out(_T - 1, 3)

    return gather_kernel


def kernel(x, W, mask):
    B = x.shape[0] * x.shape[1]
    V = W.shape[0]
    x2 = x.reshape(B // _CH, _CH)
    mf = mask.reshape(V)
    out = _build(B, V)(x2, W, mf)
    return out.reshape(x.shape[0], x.shape[1], _D)
